# baseline (speedup 1.0000x reference)
import functools

import jax
import jax.numpy as jnp
from jax import lax
from jax.experimental import pallas as pl
from jax.experimental.pallas import tpu as pltpu
from jax.experimental.pallas import tpu_sc as plsc

_BATCH = 16384

_NC = 1
_NS = 16
_NW = _NC * _NS
_B_PER_W = _BATCH // _NW

_mesh = plsc.VectorSubcoreMesh(
    core_axis_name="c", subcore_axis_name="s", num_cores=_NC
)


@functools.partial(
    pl.kernel,
    mesh=_mesh,
    out_type=jax.ShapeDtypeStruct((_BATCH,), jnp.float32),
    scratch_types=[
        pltpu.VMEM((_B_PER_W,), jnp.int32),
        pltpu.VMEM((_B_PER_W,), jnp.float32),
        pltpu.SemaphoreType.DMA,
        pltpu.SemaphoreType.DMA,
        pltpu.SemaphoreType.DMA,
    ],
)
def _sc_gather(id_hbm, data_hbm, out_hbm, idx_v, vals_v, sem_g0, sem_g1, sem_o):
    wid = lax.axis_index("s") * _NC + lax.axis_index("c")
    base = wid * _B_PER_W
    half = _B_PER_W // 2
    pltpu.sync_copy(id_hbm.at[pl.ds(base, _B_PER_W)], idx_v)
    g0 = pltpu.async_copy(
        data_hbm.at[idx_v.at[pl.ds(0, half)]], vals_v.at[pl.ds(0, half)], sem_g0
    )
    g1 = pltpu.async_copy(
        data_hbm.at[idx_v.at[pl.ds(half, half)]], vals_v.at[pl.ds(half, half)], sem_g1
    )
    g0.wait()
    o0 = pltpu.async_copy(
        vals_v.at[pl.ds(0, half)], out_hbm.at[pl.ds(base, half)], sem_o
    )
    g1.wait()
    o1 = pltpu.async_copy(
        vals_v.at[pl.ds(half, half)], out_hbm.at[pl.ds(base + half, half)], sem_o
    )
    o0.wait()
    o1.wait()


def kernel(id, data):
    return _sc_gather(id.astype(jnp.int32), data)
